# baseline (speedup 1.0000x reference)
import functools

import jax
import jax.numpy as jnp
from jax import lax
from jax.experimental import pallas as pl
from jax.experimental.pallas import tpu as pltpu
from jax.experimental.pallas import tpu_sc as plsc

VOCAB = 1000000
EMB_DIM = 32
B = 16384
L = 50

NC = 2
NS = 16
NW = NC * NS

B_TOT = B * L
B_PER_W = B_TOT // NW
CHUNK = 1600
N_CHUNKS = B_PER_W // CHUNK
N_SUB = 4

_mesh = plsc.VectorSubcoreMesh(core_axis_name="c", subcore_axis_name="s")


@functools.partial(
    pl.kernel,
    out_type=jax.ShapeDtypeStruct((B_TOT, EMB_DIM), jnp.float32),
    mesh=_mesh,
    scratch_types=[
        pltpu.VMEM((CHUNK,), jnp.int32),
        pltpu.VMEM((CHUNK,), jnp.int32),
        pltpu.VMEM((CHUNK, EMB_DIM), jnp.float32),
        pltpu.VMEM((CHUNK, EMB_DIM), jnp.float32),
        pltpu.SemaphoreType.DMA,
        pltpu.SemaphoreType.DMA,
    ],
    compiler_params=pltpu.CompilerParams(use_tc_tiling_on_sc=False),
)
def _gather_kernel(table_hbm, idx_hbm, out_hbm, i0, i1, r0, r1, s0, s1):
    wid = lax.axis_index("s") * NC + lax.axis_index("c")
    base = wid * B_PER_W
    idx_v = (i0, i1)
    rows_v = (r0, r1)
    sem = (s0, s1)

    SUB = CHUNK // N_SUB
    g = [None, None]
    for i in range(N_CHUNKS):
        b = i & 1
        off = base + i * CHUNK
        pltpu.sync_copy(idx_hbm.at[pl.ds(off, CHUNK)], idx_v[b])
        g[b] = [
            pltpu.async_copy(
                table_hbm.at[idx_v[b].at[pl.ds(j * SUB, SUB)]],
                rows_v[b].at[pl.ds(j * SUB, SUB)],
                sem[b],
            )
            for j in range(N_SUB)
        ]
        if i >= 1:
            pb = 1 - b
            for c in g[pb]:
                c.wait()
            pltpu.sync_copy(rows_v[pb], out_hbm.at[pl.ds(off - CHUNK, CHUNK)])
    lb = (N_CHUNKS - 1) & 1
    for c in g[lb]:
        c.wait()
    pltpu.sync_copy(
        rows_v[lb], out_hbm.at[pl.ds(base + (N_CHUNKS - 1) * CHUNK, CHUNK)]
    )


def kernel(batch, table):
    idx = batch.reshape(B_TOT)
    out = _gather_kernel(table, idx)
    return out.reshape(B, L, EMB_DIM)


# speedup vs baseline: 1.0041x; 1.0041x over previous
import functools

import jax
import jax.numpy as jnp
from jax import lax
from jax.experimental import pallas as pl
from jax.experimental.pallas import tpu as pltpu
from jax.experimental.pallas import tpu_sc as plsc

VOCAB = 1000000
EMB_DIM = 32
B = 16384
L = 50

NC = 2
NS = 16
NW = NC * NS

B_TOT = B * L
B_PER_W = B_TOT // NW
CHUNK = 1600
N_CHUNKS = B_PER_W // CHUNK

_mesh = plsc.VectorSubcoreMesh(core_axis_name="c", subcore_axis_name="s")


@functools.partial(
    pl.kernel,
    out_type=jax.ShapeDtypeStruct((B_TOT, EMB_DIM), jnp.float32),
    mesh=_mesh,
    scratch_types=[
        pltpu.VMEM((B_PER_W,), jnp.int32),
        pltpu.VMEM((CHUNK, EMB_DIM), jnp.float32),
        pltpu.VMEM((CHUNK, EMB_DIM), jnp.float32),
        pltpu.SemaphoreType.DMA,
        pltpu.SemaphoreType.DMA,
    ],
    compiler_params=pltpu.CompilerParams(use_tc_tiling_on_sc=False),
)
def _gather_kernel(table_hbm, idx_hbm, out_hbm, idx_v, r0, r1, s0, s1):
    wid = lax.axis_index("s") * NC + lax.axis_index("c")
    base = wid * B_PER_W
    rows_v = (r0, r1)
    sem = (s0, s1)

    pltpu.sync_copy(idx_hbm.at[pl.ds(base, B_PER_W)], idx_v)

    g = [None, None]
    for i in range(N_CHUNKS):
        b = i & 1
        g[b] = pltpu.async_copy(
            table_hbm.at[idx_v.at[pl.ds(i * CHUNK, CHUNK)]], rows_v[b], sem[b]
        )
        if i >= 1:
            pb = 1 - b
            g[pb].wait()
            pltpu.sync_copy(
                rows_v[pb], out_hbm.at[pl.ds(base + (i - 1) * CHUNK, CHUNK)]
            )
    lb = (N_CHUNKS - 1) & 1
    g[lb].wait()
    pltpu.sync_copy(
        rows_v[lb], out_hbm.at[pl.ds(base + (N_CHUNKS - 1) * CHUNK, CHUNK)]
    )


def kernel(batch, table):
    idx = batch.reshape(B_TOT)
    out = _gather_kernel(table, idx)
    return out.reshape(B, L, EMB_DIM)
